# baseline (speedup 1.0000x reference)
import functools

import jax
import jax.numpy as jnp
from jax import lax
from jax.experimental import pallas as pl
from jax.experimental.pallas import tpu as pltpu
from jax.experimental.pallas import tpu_sc as plsc

NC = 2
NS = 16
NW = NC * NS
K = 128
NP = 2


def _sc_aggregate(fh, sd, zeros, *, n, d, ew, n_acc):
    dh = d // NP
    rows_out = 1000
    n_tiles_out = n // rows_out
    ch = ew // K

    mesh = plsc.VectorSubcoreMesh(core_axis_name="c", subcore_axis_name="s")

    @functools.partial(
        pl.kernel,
        out_type=jax.ShapeDtypeStruct((NP, NC * n, dh), jnp.float32),
        mesh=mesh,
        scratch_types=[
            pltpu.VMEM_SHARED((n_acc, dh), jnp.float32),
            pltpu.VMEM_SHARED((n_acc, dh), jnp.float32),
            [pltpu.VMEM((8 * K,), jnp.int32)] * 4,
            [pltpu.VMEM((K, dh), jnp.float32)] * 2,
            [pltpu.SemaphoreType.DMA] * 2,
            [pltpu.SemaphoreType.DMA] * 2,
            [pltpu.SemaphoreType.DMA] * 4,
        ],
    )
    def sc_kernel(fh_hbm, sd_hbm, zero_hbm, out_hbm,
                  feat_s, acc, sd_v, rows_v, gsem, ssem, isem):
        c = lax.axis_index("c")
        s = lax.axis_index("s")
        wid = c * NS + s
        base2 = wid * ew * 2

        for p in range(NP):
            @pl.when(s < n_tiles_out)
            def _stage():
                pltpu.sync_copy(
                    fh_hbm.at[p, pl.ds(s * rows_out, rows_out)],
                    feat_s.at[pl.ds(s * rows_out, rows_out)])
                pltpu.sync_copy(zero_hbm,
                                acc.at[pl.ds(s * rows_out, rows_out)])
            plsc.subcore_barrier()

            nq = ch // 4

            def body(q, qp, b4, first):
                rp, rn = b4 % 2, (b4 + 1) % 2
                cur = sd_v[qp]
                nxt_set = sd_v[(qp + 1) % 4]
                if b4 == 3:
                    pltpu.make_async_copy(
                        sd_hbm.at[pl.ds(base2, 8 * K)],
                        nxt_set, isem[(qp + 1) % 4]).wait()
                g_ref = (nxt_set.at[pl.ds(0, K)] if b4 == 3 else
                         cur.at[pl.ds((b4 + 1) * 2 * K, K)])
                if not first:
                    pltpu.make_async_copy(
                        rows_v[rn], acc.at[g_ref], ssem[rn]).wait()
                pltpu.async_copy(feat_s.at[g_ref], rows_v[rn], gsem[rn])
                pltpu.make_async_copy(
                    feat_s.at[cur.at[pl.ds(b4 * 2 * K, K)]],
                    rows_v[rp], gsem[rp]).wait()
                pltpu.async_copy(
                    rows_v[rp],
                    acc.at[cur.at[pl.ds(b4 * 2 * K + K, K)]],
                    ssem[rp], add=True)
                if b4 == 0:
                    qn = jnp.minimum(q + 3, nq - 1)
                    pltpu.async_copy(
                        sd_hbm.at[pl.ds(base2 + qn * 8 * K, 8 * K)],
                        sd_v[(qp + 3) % 4], isem[(qp + 3) % 4])

            pltpu.sync_copy(sd_hbm.at[pl.ds(base2, 8 * K)], sd_v[0])
            for m in (1, 2):
                pltpu.async_copy(
                    sd_hbm.at[pl.ds(base2 + m * 8 * K, 8 * K)],
                    sd_v[m], isem[m])
            pltpu.async_copy(feat_s.at[sd_v[0].at[pl.ds(0, K)]],
                             rows_v[0], gsem[0])

            for qq in range(4):
                for b4 in range(4):
                    body(qq, qq, b4, first=(qq == 0 and b4 == 0))

            def group(g, _):
                for qq in range(4):
                    for b4 in range(4):
                        body(4 * g + qq, qq, b4, first=False)
                return ()

            lax.fori_loop(1, ch // 16, group, (), unroll=False)

            pltpu.make_async_copy(feat_s.at[sd_v[0].at[pl.ds(0, K)]],
                                  rows_v[0], gsem[0]).wait()
            pltpu.make_async_copy(rows_v[1],
                                  acc.at[sd_v[3].at[pl.ds(K, K)]],
                                  ssem[1]).wait()
            for m in (1, 2):
                pltpu.make_async_copy(sd_hbm.at[pl.ds(base2, 8 * K)],
                                      sd_v[m], isem[m]).wait()

            plsc.subcore_barrier()

            @pl.when(s < n_tiles_out)
            def _copy_out():
                pltpu.sync_copy(
                    acc.at[pl.ds(s * rows_out, rows_out)],
                    out_hbm.at[p, pl.ds(c * n + s * rows_out, rows_out)])

            plsc.subcore_barrier()

    return sc_kernel(fh, sd, zeros)


def _tc_linear(feat, hp, norm, w1, w2, b1, b2, *, n, d, blk):
    nb = n // blk
    dh = d // NP

    def body(feat_ref, h00_ref, h01_ref, h10_ref, h11_ref, norm_ref,
             w1_ref, w2_ref, b1_ref, b2_ref, out_ref):
        ah = jnp.concatenate(
            [h00_ref[0] + h01_ref[0], h10_ref[0] + h11_ref[0]],
            axis=-1) / norm_ref[...]
        dn = (((1,), (1,)), ((), ()))
        out_ref[...] = (
            lax.dot_general(feat_ref[...], w1_ref[...], dn,
                            preferred_element_type=jnp.float32)
            + lax.dot_general(ah, w2_ref[...], dn,
                              preferred_element_type=jnp.float32)
            + b1_ref[...] + b2_ref[...])

    return pl.pallas_call(
        body,
        grid=(nb,),
        in_specs=[
            pl.BlockSpec((blk, d), lambda i: (i, 0)),
            pl.BlockSpec((1, blk, dh), lambda i: (0, i, 0)),
            pl.BlockSpec((1, blk, dh), lambda i: (0, i + nb, 0)),
            pl.BlockSpec((1, blk, dh), lambda i: (1, i, 0)),
            pl.BlockSpec((1, blk, dh), lambda i: (1, i + nb, 0)),
            pl.BlockSpec((blk, 1), lambda i: (i, 0)),
            pl.BlockSpec((d, d), lambda i: (0, 0)),
            pl.BlockSpec((d, d), lambda i: (0, 0)),
            pl.BlockSpec((1, d), lambda i: (0, 0)),
            pl.BlockSpec((1, d), lambda i: (0, 0)),
        ],
        out_specs=pl.BlockSpec((blk, d), lambda i: (i, 0)),
        out_shape=jax.ShapeDtypeStruct((n, d), jnp.float32),
    )(feat, hp, hp, hp, hp, norm, w1, w2, b1, b2)


def kernel(feat, edge_index, in_norm, W1, b1, W2, b2):
    n, d = feat.shape
    e = edge_index.shape[1]
    dh = d // NP

    ew = -(-e // (NW * K * 16)) * (K * 16)
    pad = NW * ew - e
    src = jnp.concatenate([edge_index[0],
                           jnp.zeros((pad,), jnp.int32)])
    dst = jnp.concatenate([edge_index[1],
                           jnp.full((pad,), n, jnp.int32)])

    rows_z = -(-(n + 1) // (NS * 8)) * 8
    n_pad = rows_z * NS

    fh = jnp.stack([feat[:, p * dh:(p + 1) * dh] for p in range(NP)])
    zeros = jnp.zeros((1000, dh), jnp.float32)

    sd = jnp.concatenate([src.reshape(-1, 1, K), dst.reshape(-1, 1, K)],
                         axis=1).reshape(-1)
    hp = _sc_aggregate(fh, sd, zeros, n=n, d=d, ew=ew, n_acc=n_pad)
    return _tc_linear(feat, hp, in_norm[:, None], W1, W2,
                      b1[None, :], b2[None, :], n=n, d=d, blk=1000)


# speedup vs baseline: 1.0013x; 1.0013x over previous
import functools

import jax
import jax.numpy as jnp
from jax import lax
from jax.experimental import pallas as pl
from jax.experimental.pallas import tpu as pltpu
from jax.experimental.pallas import tpu_sc as plsc

NC = 2
NS = 16
NW = NC * NS
K = 128
NP = 2


def _sc_aggregate(fh, sd, zeros, *, n, d, ew, n_acc):
    dh = d // NP
    rows_out = 1000
    n_tiles_out = n // rows_out
    ch = ew // K

    mesh = plsc.VectorSubcoreMesh(core_axis_name="c", subcore_axis_name="s")

    @functools.partial(
        pl.kernel,
        out_type=jax.ShapeDtypeStruct((NP, NC * n, dh), jnp.float32),
        mesh=mesh,
        scratch_types=[
            pltpu.VMEM_SHARED((n_acc, dh), jnp.float32),
            pltpu.VMEM_SHARED((n_acc, dh), jnp.float32),
            [pltpu.VMEM((2 * K,), jnp.int32)] * 4,
            [pltpu.VMEM((K, dh), jnp.float32)] * 2,
            [pltpu.SemaphoreType.DMA] * 2,
            [pltpu.SemaphoreType.DMA] * 2,
            [pltpu.SemaphoreType.DMA] * 4,
        ],
    )
    def sc_kernel(fh_hbm, sd_hbm, zero_hbm, out_hbm,
                  feat_s, acc, sd_v, rows_v, gsem, ssem, isem):
        c = lax.axis_index("c")
        s = lax.axis_index("s")
        wid = c * NS + s
        base2 = wid * ew * 2

        for p in range(NP):
            @pl.when(s < n_tiles_out)
            def _stage():
                pltpu.sync_copy(
                    fh_hbm.at[p, pl.ds(s * rows_out, rows_out)],
                    feat_s.at[pl.ds(s * rows_out, rows_out)])
                pltpu.sync_copy(zero_hbm,
                                acc.at[pl.ds(s * rows_out, rows_out)])
            plsc.subcore_barrier()

            def body(j0, b4, first):
                rp, rn = b4 % 2, (b4 + 1) % 2
                mN, mP = (b4 + 1) % 4, (b4 + 3) % 4
                pltpu.make_async_copy(sd_hbm.at[pl.ds(base2, 2 * K)],
                                      sd_v[mN], isem[mN]).wait()
                if not first:
                    pltpu.make_async_copy(
                        rows_v[rn], acc.at[sd_v[mN].at[pl.ds(K, K)]],
                        ssem[rn]).wait()
                pltpu.async_copy(feat_s.at[sd_v[mN].at[pl.ds(0, K)]],
                                 rows_v[rn], gsem[rn])
                pltpu.make_async_copy(feat_s.at[sd_v[b4].at[pl.ds(0, K)]],
                                      rows_v[rp], gsem[rp]).wait()
                pltpu.async_copy(rows_v[rp],
                                 acc.at[sd_v[b4].at[pl.ds(K, K)]],
                                 ssem[rp], add=True)
                nxt = base2 + jnp.minimum(j0 + b4 + 3, ch - 1) * 2 * K
                pltpu.async_copy(sd_hbm.at[pl.ds(nxt, 2 * K)],
                                 sd_v[mP], isem[mP])

            pltpu.sync_copy(sd_hbm.at[pl.ds(base2, 2 * K)], sd_v[0])
            for m in (1, 2):
                pltpu.async_copy(
                    sd_hbm.at[pl.ds(base2 + m * 2 * K, 2 * K)],
                    sd_v[m], isem[m])
            pltpu.async_copy(feat_s.at[sd_v[0].at[pl.ds(0, K)]],
                             rows_v[0], gsem[0])

            for b4 in range(4):
                body(0, b4, first=(b4 == 0))

            def quad(t, _):
                for b4 in range(4):
                    body(4 * t, b4, first=False)
                return ()

            lax.fori_loop(1, ch // 4, quad, (), unroll=False)

            pltpu.make_async_copy(feat_s.at[sd_v[0].at[pl.ds(0, K)]],
                                  rows_v[0], gsem[0]).wait()
            pltpu.make_async_copy(rows_v[1],
                                  acc.at[sd_v[3].at[pl.ds(K, K)]],
                                  ssem[1]).wait()
            for m in (1, 2):
                pltpu.make_async_copy(sd_hbm.at[pl.ds(base2, 2 * K)],
                                      sd_v[m], isem[m]).wait()

            plsc.subcore_barrier()

            @pl.when(s < n_tiles_out)
            def _copy_out():
                pltpu.sync_copy(
                    acc.at[pl.ds(s * rows_out, rows_out)],
                    out_hbm.at[p, pl.ds(c * n + s * rows_out, rows_out)])

            plsc.subcore_barrier()

    return sc_kernel(fh, sd, zeros)


def _tc_first(feat, w1, b1, b2, *, n, d, blk):
    nb = n // blk

    def body(feat_ref, w1_ref, b1_ref, b2_ref, out_ref):
        dn = (((1,), (1,)), ((), ()))
        out_ref[...] = (
            lax.dot_general(feat_ref[...], w1_ref[...], dn,
                            preferred_element_type=jnp.float32)
            + b1_ref[...] + b2_ref[...])

    return pl.pallas_call(
        body,
        grid=(nb,),
        in_specs=[
            pl.BlockSpec((blk, d), lambda i: (i, 0)),
            pl.BlockSpec((d, d), lambda i: (0, 0)),
            pl.BlockSpec((1, d), lambda i: (0, 0)),
            pl.BlockSpec((1, d), lambda i: (0, 0)),
        ],
        out_specs=pl.BlockSpec((blk, d), lambda i: (i, 0)),
        out_shape=jax.ShapeDtypeStruct((n, d), jnp.float32),
    )(feat, w1, b1, b2)


def _tc_linear(t1, hp, norm, w2, *, n, d, blk):
    nb = n // blk
    dh = d // NP

    def body(t1_ref, h00_ref, h01_ref, h10_ref, h11_ref, norm_ref,
             w2_ref, out_ref):
        ah = jnp.concatenate(
            [h00_ref[0] + h01_ref[0], h10_ref[0] + h11_ref[0]],
            axis=-1) / norm_ref[...]
        dn = (((1,), (1,)), ((), ()))
        out_ref[...] = t1_ref[...] + lax.dot_general(
            ah, w2_ref[...], dn, preferred_element_type=jnp.float32)

    return pl.pallas_call(
        body,
        grid=(nb,),
        in_specs=[
            pl.BlockSpec((blk, d), lambda i: (i, 0)),
            pl.BlockSpec((1, blk, dh), lambda i: (0, i, 0)),
            pl.BlockSpec((1, blk, dh), lambda i: (0, i + nb, 0)),
            pl.BlockSpec((1, blk, dh), lambda i: (1, i, 0)),
            pl.BlockSpec((1, blk, dh), lambda i: (1, i + nb, 0)),
            pl.BlockSpec((blk, 1), lambda i: (i, 0)),
            pl.BlockSpec((d, d), lambda i: (0, 0)),
        ],
        out_specs=pl.BlockSpec((blk, d), lambda i: (i, 0)),
        out_shape=jax.ShapeDtypeStruct((n, d), jnp.float32),
    )(t1, hp, hp, hp, hp, norm, w2)


def kernel(feat, edge_index, in_norm, W1, b1, W2, b2):
    n, d = feat.shape
    e = edge_index.shape[1]
    dh = d // NP

    ew = -(-e // (NW * K * 4)) * (K * 4)
    pad = NW * ew - e
    src = jnp.concatenate([edge_index[0],
                           jnp.zeros((pad,), jnp.int32)])
    dst = jnp.concatenate([edge_index[1],
                           jnp.full((pad,), n, jnp.int32)])

    rows_z = -(-(n + 1) // (NS * 8)) * 8
    n_pad = rows_z * NS

    fh = jnp.stack([feat[:, p * dh:(p + 1) * dh] for p in range(NP)])
    zeros = jnp.zeros((1000, dh), jnp.float32)

    sd = jnp.concatenate([src.reshape(-1, 1, K), dst.reshape(-1, 1, K)],
                         axis=1).reshape(-1)
    t1 = _tc_first(feat, W1, b1[None, :], b2[None, :], n=n, d=d, blk=1000)
    hp = _sc_aggregate(fh, sd, zeros, n=n, d=d, ew=ew, n_acc=n_pad)
    return _tc_linear(t1, hp, in_norm[:, None], W2, n=n, d=d, blk=1000)


# speedup vs baseline: 1.0106x; 1.0093x over previous
import functools

import jax
import jax.numpy as jnp
from jax import lax
from jax.experimental import pallas as pl
from jax.experimental.pallas import tpu as pltpu
from jax.experimental.pallas import tpu_sc as plsc

NC = 2
NS = 16
NW = NC * NS
K = 128
NP = 2


def _sc_aggregate(fh, sd, zeros, *, n, d, ew, n_acc):
    dh = d // NP
    rows_out = 1000
    n_tiles_out = n // rows_out
    ch = ew // K

    mesh = plsc.VectorSubcoreMesh(core_axis_name="c", subcore_axis_name="s")

    @functools.partial(
        pl.kernel,
        out_type=jax.ShapeDtypeStruct((NP, NC * n, dh), jnp.float32),
        mesh=mesh,
        scratch_types=[
            pltpu.VMEM_SHARED((n_acc, dh), jnp.float32),
            pltpu.VMEM_SHARED((n_acc, dh), jnp.float32),
            [pltpu.VMEM((2 * K,), jnp.int32)] * 4,
            [pltpu.VMEM((K, dh), jnp.float32)] * 2,
            [pltpu.SemaphoreType.DMA] * 2,
            [pltpu.SemaphoreType.DMA] * 2,
            [pltpu.SemaphoreType.DMA] * 4,
        ],
    )
    def sc_kernel(fh_hbm, sd_hbm, zero_hbm, out_hbm,
                  feat_s, acc, sd_v, rows_v, gsem, ssem, isem):
        c = lax.axis_index("c")
        s = lax.axis_index("s")
        wid = c * NS + s
        base2 = wid * ew * 2

        for p in range(NP):
            @pl.when(s < n_tiles_out)
            def _stage():
                pltpu.sync_copy(
                    fh_hbm.at[p, pl.ds(s * rows_out, rows_out)],
                    feat_s.at[pl.ds(s * rows_out, rows_out)])
                pltpu.sync_copy(zero_hbm,
                                acc.at[pl.ds(s * rows_out, rows_out)])
            plsc.subcore_barrier()

            def body(j0, b4, first):
                rp, rn = b4 % 2, (b4 + 1) % 2
                mN, mP = (b4 + 1) % 4, (b4 + 3) % 4
                pltpu.make_async_copy(sd_hbm.at[pl.ds(base2, 2 * K)],
                                      sd_v[mN], isem[mN]).wait()
                if not first:
                    pltpu.make_async_copy(
                        rows_v[rn], acc.at[sd_v[mN].at[pl.ds(K, K)]],
                        ssem[rn]).wait()
                pltpu.async_copy(feat_s.at[sd_v[mN].at[pl.ds(0, K)]],
                                 rows_v[rn], gsem[rn])
                pltpu.make_async_copy(feat_s.at[sd_v[b4].at[pl.ds(0, K)]],
                                      rows_v[rp], gsem[rp]).wait()
                pltpu.async_copy(rows_v[rp],
                                 acc.at[sd_v[b4].at[pl.ds(K, K)]],
                                 ssem[rp], add=True)
                nxt = base2 + jnp.minimum(j0 + b4 + 3, ch - 1) * 2 * K
                pltpu.async_copy(sd_hbm.at[pl.ds(nxt, 2 * K)],
                                 sd_v[mP], isem[mP])

            pltpu.sync_copy(sd_hbm.at[pl.ds(base2, 2 * K)], sd_v[0])
            for m in (1, 2):
                pltpu.async_copy(
                    sd_hbm.at[pl.ds(base2 + m * 2 * K, 2 * K)],
                    sd_v[m], isem[m])
            pltpu.async_copy(feat_s.at[sd_v[0].at[pl.ds(0, K)]],
                             rows_v[0], gsem[0])

            for b4 in range(4):
                body(0, b4, first=(b4 == 0))

            def quad(t, _):
                for b4 in range(4):
                    body(4 * t, b4, first=False)
                return ()

            lax.fori_loop(1, ch // 4, quad, (), unroll=False)

            pltpu.make_async_copy(feat_s.at[sd_v[0].at[pl.ds(0, K)]],
                                  rows_v[0], gsem[0]).wait()
            pltpu.make_async_copy(rows_v[1],
                                  acc.at[sd_v[3].at[pl.ds(K, K)]],
                                  ssem[1]).wait()
            for m in (1, 2):
                pltpu.make_async_copy(sd_hbm.at[pl.ds(base2, 2 * K)],
                                      sd_v[m], isem[m]).wait()

            plsc.subcore_barrier()

            @pl.when(s < n_tiles_out)
            def _copy_out():
                pltpu.sync_copy(
                    acc.at[pl.ds(s * rows_out, rows_out)],
                    out_hbm.at[p, pl.ds(c * n + s * rows_out, rows_out)])

            plsc.subcore_barrier()

    return sc_kernel(fh, sd, zeros)


def _tc_linear(feat, hp, norm, w1, w2, b1, b2, *, n, d, blk):
    nb = n // blk
    dh = d // NP

    def body(feat_ref, h00_ref, h01_ref, h10_ref, h11_ref, norm_ref,
             w1_ref, w2_ref, b1_ref, b2_ref, out_ref):
        ah = jnp.concatenate(
            [h00_ref[0] + h01_ref[0], h10_ref[0] + h11_ref[0]],
            axis=-1) / norm_ref[...]
        dn = (((1,), (1,)), ((), ()))
        out_ref[...] = (
            lax.dot_general(feat_ref[...], w1_ref[...], dn,
                            preferred_element_type=jnp.float32)
            + lax.dot_general(ah, w2_ref[...], dn,
                              preferred_element_type=jnp.float32)
            + b1_ref[...] + b2_ref[...])

    return pl.pallas_call(
        body,
        grid=(nb,),
        in_specs=[
            pl.BlockSpec((blk, d), lambda i: (i, 0)),
            pl.BlockSpec((1, blk, dh), lambda i: (0, i, 0)),
            pl.BlockSpec((1, blk, dh), lambda i: (0, i + nb, 0)),
            pl.BlockSpec((1, blk, dh), lambda i: (1, i, 0)),
            pl.BlockSpec((1, blk, dh), lambda i: (1, i + nb, 0)),
            pl.BlockSpec((blk, 1), lambda i: (i, 0)),
            pl.BlockSpec((d, d), lambda i: (0, 0)),
            pl.BlockSpec((d, d), lambda i: (0, 0)),
            pl.BlockSpec((1, d), lambda i: (0, 0)),
            pl.BlockSpec((1, d), lambda i: (0, 0)),
        ],
        out_specs=pl.BlockSpec((blk, d), lambda i: (i, 0)),
        out_shape=jax.ShapeDtypeStruct((n, d), jnp.float32),
    )(feat, hp, hp, hp, hp, norm, w1, w2, b1, b2)


def kernel(feat, edge_index, in_norm, W1, b1, W2, b2):
    n, d = feat.shape
    e = edge_index.shape[1]
    dh = d // NP

    ew = -(-e // (NW * K * 4)) * (K * 4)
    pad = NW * ew - e
    src = jnp.concatenate([edge_index[0],
                           jnp.zeros((pad,), jnp.int32)])
    dst = jnp.concatenate([edge_index[1],
                           jnp.full((pad,), n, jnp.int32)])

    rows_z = -(-(n + 1) // (NS * 8)) * 8
    n_pad = rows_z * NS

    fh = jnp.stack([feat[:, p * dh:(p + 1) * dh] for p in range(NP)])
    zeros = jnp.zeros((1000, dh), jnp.float32)

    sd = jnp.concatenate([src.reshape(-1, 1, K), dst.reshape(-1, 1, K)],
                         axis=1).reshape(-1)
    hp = _sc_aggregate(fh, sd, zeros, n=n, d=d, ew=ew, n_acc=n_pad)
    return _tc_linear(feat, hp, in_norm[:, None], W1, W2,
                      b1[None, :], b2[None, :], n=n, d=d, blk=2000)
